# Initial kernel scaffold; baseline (speedup 1.0000x reference)
import jax
import jax.numpy as jnp
from jax.experimental import pallas as pl


def kernel(positions, mapping):
    raise NotImplementedError("write your pallas kernel here")


# speedup vs baseline: 48.3842x; 48.3842x over previous
import jax
import jax.numpy as jnp
from jax import lax
from jax.experimental import pallas as pl
from jax.experimental.pallas import tpu as pltpu
from jax.experimental.pallas import tpu_sc as plsc

_SIGMA = 1.0
_EPSILON = 5.0
_ALPHA = 5.0
_CUTOFF = 2.5

_NC = 2
_NS = 16
_NW = _NC * _NS
_L = 16
_C = 2000
_NPAD = 100096


def _rsqrt(d2):
    i = lax.bitcast_convert_type(d2, jnp.int32)
    i = jnp.int32(0x5F3759DF) - (i >> 1)
    y = lax.bitcast_convert_type(i, jnp.float32)
    for _ in range(3):
        y = y * (1.5 - 0.5 * d2 * y * y)
    return y


def _sc_body(px_hbm, py_hbm, pz_hbm, src_hbm, dst_hbm,
             ae_out, fx_out, fy_out, fz_out, esum_out,
             idx_s, idx_d,
             gsx, gsy, gsz, gdx, gdy, gdz,
             ue, usx, usy, usz, udx, udy, udz, esum_v, stage,
             xs_sh, ys_sh, zs_sh, ae_sh, fx_sh, fy_sh, fz_sh):
    npad = px_hbm.shape[0]
    e = src_hbm.shape[0]
    cid = lax.axis_index("c")
    sid = lax.axis_index("s")
    wid = sid * _NC + cid
    rpt = npad // _NS
    r0 = sid * rpt

    sl = pl.ds(r0, rpt)

    def zfill(j, _):
        stage[pl.ds(j * _L, _L)] = jnp.zeros((_L,), jnp.float32)
        return 0

    lax.fori_loop(0, rpt // _L, zfill, 0)
    pltpu.sync_copy(stage, ae_sh.at[sl])
    pltpu.sync_copy(stage, fx_sh.at[sl])
    pltpu.sync_copy(stage, fy_sh.at[sl])
    pltpu.sync_copy(stage, fz_sh.at[sl])
    for hbm, sh in ((px_hbm, xs_sh), (py_hbm, ys_sh), (pz_hbm, zs_sh)):
        pltpu.sync_copy(hbm.at[sl], stage)
        pltpu.sync_copy(stage, sh.at[sl])
    plsc.subcore_barrier()

    e_per_w = e // _NW
    n_chunks = e_per_w // _C

    def grp(g, ev):
        s = pl.ds(g * _L, _L)
        xs = gsx[s]
        ys = gsy[s]
        zs = gsz[s]
        xd = gdx[s]
        yd = gdy[s]
        zd = gdz[s]
        dx = xd - xs
        dy = yd - ys
        dz = zd - zs
        d2 = dx * dx + dy * dy + dz * dz
        valid = d2 > 0.0
        yk = _rsqrt(d2)
        dr = jnp.where(valid, d2 * yk, 0.0)
        em = jnp.exp(-_ALPHA * (dr - _SIGMA))
        om = 1.0 - em
        mask = valid & (dr < _CUTOFF)
        pe = jnp.where(mask, _EPSILON * om * om - _EPSILON, 0.0)
        pf = jnp.where(mask, (-2.0 * _ALPHA * _EPSILON) * em * om, 0.0)
        sc = pf * yk
        fx = sc * dx
        fy = sc * dy
        fz = sc * dz
        ue[s] = 0.5 * pe
        usx[s] = -fx
        usy[s] = -fy
        usz[s] = -fz
        udx[s] = fx
        udy[s] = fy
        udz[s] = fz
        return ev + pe

    def chunk(i, ev):
        base = wid * e_per_w + i * _C
        pltpu.sync_copy(src_hbm.at[pl.ds(base, _C)], idx_s)
        pltpu.sync_copy(dst_hbm.at[pl.ds(base, _C)], idx_d)
        pltpu.sync_copy(xs_sh.at[idx_s], gsx)
        pltpu.sync_copy(ys_sh.at[idx_s], gsy)
        pltpu.sync_copy(zs_sh.at[idx_s], gsz)
        pltpu.sync_copy(xs_sh.at[idx_d], gdx)
        pltpu.sync_copy(ys_sh.at[idx_d], gdy)
        pltpu.sync_copy(zs_sh.at[idx_d], gdz)
        ev = lax.fori_loop(0, _C // _L, grp, ev)
        pltpu.sync_copy(ue, ae_sh.at[idx_s], add=True)
        pltpu.sync_copy(ue, ae_sh.at[idx_d], add=True)
        pltpu.sync_copy(usx, fx_sh.at[idx_s], add=True)
        pltpu.sync_copy(usy, fy_sh.at[idx_s], add=True)
        pltpu.sync_copy(usz, fz_sh.at[idx_s], add=True)
        pltpu.sync_copy(udx, fx_sh.at[idx_d], add=True)
        pltpu.sync_copy(udy, fy_sh.at[idx_d], add=True)
        pltpu.sync_copy(udz, fz_sh.at[idx_d], add=True)
        return ev

    evec = lax.fori_loop(0, n_chunks, chunk, jnp.zeros((_L,), jnp.float32))
    esum_v[...] = evec
    plsc.subcore_barrier()
    osl = pl.ds(cid * npad + r0, rpt)
    for sh, out in ((ae_sh, ae_out), (fx_sh, fx_out),
                    (fy_sh, fy_out), (fz_sh, fz_out)):
        pltpu.sync_copy(sh.at[sl], stage)
        pltpu.sync_copy(stage, out.at[osl])
    pltpu.sync_copy(esum_v, esum_out.at[pl.ds(wid * _L, _L)])


def kernel(positions, mapping):
    n = positions.shape[0]
    pad = _NPAD - n
    px = jnp.pad(positions[:, 0], (0, pad))
    py = jnp.pad(positions[:, 1], (0, pad))
    pz = jnp.pad(positions[:, 2], (0, pad))
    src = mapping[0]
    dst = mapping[1]

    mesh = plsc.VectorSubcoreMesh(core_axis_name="c", subcore_axis_name="s")
    f32 = jnp.float32
    kfn = pl.kernel(
        _sc_body,
        out_type=(
            jax.ShapeDtypeStruct((_NC * _NPAD,), f32),
            jax.ShapeDtypeStruct((_NC * _NPAD,), f32),
            jax.ShapeDtypeStruct((_NC * _NPAD,), f32),
            jax.ShapeDtypeStruct((_NC * _NPAD,), f32),
            jax.ShapeDtypeStruct((_NW * _L,), f32),
        ),
        mesh=mesh,
        scratch_types=(
            [pltpu.VMEM((_C,), jnp.int32)] * 2
            + [pltpu.VMEM((_C,), f32)] * 6
            + [pltpu.VMEM((_C,), f32)] * 7
            + [pltpu.VMEM((_L,), f32)]
            + [pltpu.VMEM((_NPAD // _NS,), f32)]
            + [pltpu.VMEM_SHARED((_NPAD,), f32)] * 7
        ),
    )
    ae, fx, fy, fz, esum = kfn(px, py, pz, src, dst)
    atom_energies = ae[:n] + ae[_NPAD:_NPAD + n]
    forces = jnp.stack(
        [fx[:n] + fx[_NPAD:_NPAD + n],
         fy[:n] + fy[_NPAD:_NPAD + n],
         fz[:n] + fz[_NPAD:_NPAD + n]], axis=-1)
    energy = 0.5 * jnp.sum(esum)
    return energy, atom_energies, forces


# speedup vs baseline: 68.8961x; 1.4239x over previous
import jax
import jax.numpy as jnp
from jax import lax
from jax.experimental import pallas as pl
from jax.experimental.pallas import tpu as pltpu
from jax.experimental.pallas import tpu_sc as plsc

_SIGMA = 1.0
_EPSILON = 5.0
_ALPHA = 5.0
_CUTOFF = 2.5

_NC = 2
_NS = 16
_NW = _NC * _NS
_L = 16
_C = 2000
_NPAD = 100096


def _rsqrt(d2):
    i = lax.bitcast_convert_type(d2, jnp.int32)
    i = jnp.int32(0x5F3759DF) - (i >> 1)
    y = lax.bitcast_convert_type(i, jnp.float32)
    for _ in range(3):
        y = y * (1.5 - 0.5 * d2 * y * y)
    return y


def _sc_body(px_hbm, py_hbm, pz_hbm, src_hbm, dst_hbm,
             ae_out, fx_out, fy_out, fz_out, esum_out,
             idx_s, idx_d,
             gsx, gsy, gsz, gdx, gdy, gdz,
             ue, usx, usy, usz, udx, udy, udz, esum_v, stage,
             sem_i, sem_g, sem_s,
             xs_sh, ys_sh, zs_sh, ae_sh, fx_sh, fy_sh, fz_sh):
    npad = px_hbm.shape[0]
    e = src_hbm.shape[0]
    cid = lax.axis_index("c")
    sid = lax.axis_index("s")
    wid = sid * _NC + cid
    rpt = npad // _NS
    r0 = sid * rpt

    sl = pl.ds(r0, rpt)

    def zfill(j, _):
        stage[pl.ds(j * _L, _L)] = jnp.zeros((_L,), jnp.float32)
        return 0

    lax.fori_loop(0, rpt // _L, zfill, 0)
    pltpu.sync_copy(stage, ae_sh.at[sl])
    pltpu.sync_copy(stage, fx_sh.at[sl])
    pltpu.sync_copy(stage, fy_sh.at[sl])
    pltpu.sync_copy(stage, fz_sh.at[sl])
    for hbm, sh in ((px_hbm, xs_sh), (py_hbm, ys_sh), (pz_hbm, zs_sh)):
        pltpu.sync_copy(hbm.at[sl], stage)
        pltpu.sync_copy(stage, sh.at[sl])
    plsc.subcore_barrier()

    e_per_w = e // _NW
    n_chunks = e_per_w // _C

    def grp(g, ev):
        s = pl.ds(g * _L, _L)
        xs = gsx[s]
        ys = gsy[s]
        zs = gsz[s]
        xd = gdx[s]
        yd = gdy[s]
        zd = gdz[s]
        dx = xd - xs
        dy = yd - ys
        dz = zd - zs
        d2 = dx * dx + dy * dy + dz * dz
        valid = d2 > 0.0
        yk = _rsqrt(d2)
        dr = jnp.where(valid, d2 * yk, 0.0)
        em = jnp.exp(-_ALPHA * (dr - _SIGMA))
        om = 1.0 - em
        mask = valid & (dr < _CUTOFF)
        pe = jnp.where(mask, _EPSILON * om * om - _EPSILON, 0.0)
        pf = jnp.where(mask, (-2.0 * _ALPHA * _EPSILON) * em * om, 0.0)
        sc = pf * yk
        fx = sc * dx
        fy = sc * dy
        fz = sc * dz
        ue[s] = 0.5 * pe
        usx[s] = -fx
        usy[s] = -fy
        usz[s] = -fz
        udx[s] = fx
        udy[s] = fy
        udz[s] = fz
        return ev + pe

    def chunk(i, ev):
        base = wid * e_per_w + i * _C
        d1 = pltpu.async_copy(src_hbm.at[pl.ds(base, _C)], idx_s, sem_i)
        d2 = pltpu.async_copy(dst_hbm.at[pl.ds(base, _C)], idx_d, sem_i)
        d1.wait()
        d2.wait()
        gds = [
            pltpu.async_copy(xs_sh.at[idx_s], gsx, sem_g),
            pltpu.async_copy(ys_sh.at[idx_s], gsy, sem_g),
            pltpu.async_copy(zs_sh.at[idx_s], gsz, sem_g),
            pltpu.async_copy(xs_sh.at[idx_d], gdx, sem_g),
            pltpu.async_copy(ys_sh.at[idx_d], gdy, sem_g),
            pltpu.async_copy(zs_sh.at[idx_d], gdz, sem_g),
        ]
        for d in gds:
            d.wait()
        ev = lax.fori_loop(0, _C // _L, grp, ev)
        sds = [
            pltpu.async_copy(ue, ae_sh.at[idx_s], sem_s, add=True),
            pltpu.async_copy(ue, ae_sh.at[idx_d], sem_s, add=True),
            pltpu.async_copy(usx, fx_sh.at[idx_s], sem_s, add=True),
            pltpu.async_copy(usy, fy_sh.at[idx_s], sem_s, add=True),
            pltpu.async_copy(usz, fz_sh.at[idx_s], sem_s, add=True),
            pltpu.async_copy(udx, fx_sh.at[idx_d], sem_s, add=True),
            pltpu.async_copy(udy, fy_sh.at[idx_d], sem_s, add=True),
            pltpu.async_copy(udz, fz_sh.at[idx_d], sem_s, add=True),
        ]
        for d in sds:
            d.wait()
        return ev

    evec = lax.fori_loop(0, n_chunks, chunk, jnp.zeros((_L,), jnp.float32))
    esum_v[...] = evec
    plsc.subcore_barrier()
    osl = pl.ds(cid * npad + r0, rpt)
    for sh, out in ((ae_sh, ae_out), (fx_sh, fx_out),
                    (fy_sh, fy_out), (fz_sh, fz_out)):
        pltpu.sync_copy(sh.at[sl], stage)
        pltpu.sync_copy(stage, out.at[osl])
    pltpu.sync_copy(esum_v, esum_out.at[pl.ds(wid * _L, _L)])


def kernel(positions, mapping):
    n = positions.shape[0]
    pad = _NPAD - n
    px = jnp.pad(positions[:, 0], (0, pad))
    py = jnp.pad(positions[:, 1], (0, pad))
    pz = jnp.pad(positions[:, 2], (0, pad))
    src = mapping[0]
    dst = mapping[1]

    mesh = plsc.VectorSubcoreMesh(core_axis_name="c", subcore_axis_name="s")
    f32 = jnp.float32
    kfn = pl.kernel(
        _sc_body,
        out_type=(
            jax.ShapeDtypeStruct((_NC * _NPAD,), f32),
            jax.ShapeDtypeStruct((_NC * _NPAD,), f32),
            jax.ShapeDtypeStruct((_NC * _NPAD,), f32),
            jax.ShapeDtypeStruct((_NC * _NPAD,), f32),
            jax.ShapeDtypeStruct((_NW * _L,), f32),
        ),
        mesh=mesh,
        scratch_types=(
            [pltpu.VMEM((_C,), jnp.int32)] * 2
            + [pltpu.VMEM((_C,), f32)] * 6
            + [pltpu.VMEM((_C,), f32)] * 7
            + [pltpu.VMEM((_L,), f32)]
            + [pltpu.VMEM((_NPAD // _NS,), f32)]
            + [pltpu.SemaphoreType.DMA] * 3
            + [pltpu.VMEM_SHARED((_NPAD,), f32)] * 7
        ),
    )
    ae, fx, fy, fz, esum = kfn(px, py, pz, src, dst)
    atom_energies = ae[:n] + ae[_NPAD:_NPAD + n]
    forces = jnp.stack(
        [fx[:n] + fx[_NPAD:_NPAD + n],
         fy[:n] + fy[_NPAD:_NPAD + n],
         fz[:n] + fz[_NPAD:_NPAD + n]], axis=-1)
    energy = 0.5 * jnp.sum(esum)
    return energy, atom_energies, forces


# speedup vs baseline: 73.1921x; 1.0624x over previous
import jax
import jax.numpy as jnp
from jax import lax
from jax.experimental import pallas as pl
from jax.experimental.pallas import tpu as pltpu
from jax.experimental.pallas import tpu_sc as plsc

_SIGMA = 1.0
_EPSILON = 5.0
_ALPHA = 5.0
_CUTOFF = 2.5

_NC = 2
_NS = 16
_NW = _NC * _NS
_L = 16
_C = 4000
_NPAD = 100096


def _rsqrt(d2):
    i = lax.bitcast_convert_type(d2, jnp.int32)
    i = jnp.int32(0x5F3759DF) - (i >> 1)
    y = lax.bitcast_convert_type(i, jnp.float32)
    for _ in range(3):
        y = y * (1.5 - 0.5 * d2 * y * y)
    return y


def _sc_body(px_hbm, py_hbm, pz_hbm, src_hbm, dst_hbm,
             ae_out, fx_out, fy_out, fz_out, esum_out,
             idx_s, idx_d,
             gsx, gsy, gsz, gdx, gdy, gdz,
             ue, usx, usy, usz, udx, udy, udz, esum_v, stage,
             sem_i, sem_g, sem_s,
             xs_sh, ys_sh, zs_sh, ae_sh, fx_sh, fy_sh, fz_sh):
    npad = px_hbm.shape[0]
    e = src_hbm.shape[0]
    cid = lax.axis_index("c")
    sid = lax.axis_index("s")
    wid = sid * _NC + cid
    rpt = npad // _NS
    r0 = sid * rpt

    sl = pl.ds(r0, rpt)

    def zfill(j, _):
        stage[pl.ds(j * _L, _L)] = jnp.zeros((_L,), jnp.float32)
        return 0

    lax.fori_loop(0, rpt // _L, zfill, 0)
    pltpu.sync_copy(stage, ae_sh.at[sl])
    pltpu.sync_copy(stage, fx_sh.at[sl])
    pltpu.sync_copy(stage, fy_sh.at[sl])
    pltpu.sync_copy(stage, fz_sh.at[sl])
    for hbm, sh in ((px_hbm, xs_sh), (py_hbm, ys_sh), (pz_hbm, zs_sh)):
        pltpu.sync_copy(hbm.at[sl], stage)
        pltpu.sync_copy(stage, sh.at[sl])
    plsc.subcore_barrier()

    e_per_w = e // _NW
    n_chunks = e_per_w // _C

    def grp(g, ev):
        s = pl.ds(g * _L, _L)
        xs = gsx[s]
        ys = gsy[s]
        zs = gsz[s]
        xd = gdx[s]
        yd = gdy[s]
        zd = gdz[s]
        dx = xd - xs
        dy = yd - ys
        dz = zd - zs
        d2 = dx * dx + dy * dy + dz * dz
        valid = d2 > 0.0
        yk = _rsqrt(d2)
        dr = jnp.where(valid, d2 * yk, 0.0)
        em = jnp.exp(-_ALPHA * (dr - _SIGMA))
        om = 1.0 - em
        mask = valid & (dr < _CUTOFF)
        pe = jnp.where(mask, _EPSILON * om * om - _EPSILON, 0.0)
        pf = jnp.where(mask, (-2.0 * _ALPHA * _EPSILON) * em * om, 0.0)
        sc = pf * yk
        fx = sc * dx
        fy = sc * dy
        fz = sc * dz
        ue[s] = 0.5 * pe
        usx[s] = -fx
        usy[s] = -fy
        usz[s] = -fz
        udx[s] = fx
        udy[s] = fy
        udz[s] = fz
        return ev + pe

    def chunk(i, ev):
        base = wid * e_per_w + i * _C
        d1 = pltpu.async_copy(src_hbm.at[pl.ds(base, _C)], idx_s, sem_i)
        d2 = pltpu.async_copy(dst_hbm.at[pl.ds(base, _C)], idx_d, sem_i)
        d1.wait()
        d2.wait()
        gds = [
            pltpu.async_copy(xs_sh.at[idx_s], gsx, sem_g),
            pltpu.async_copy(ys_sh.at[idx_s], gsy, sem_g),
            pltpu.async_copy(zs_sh.at[idx_s], gsz, sem_g),
            pltpu.async_copy(xs_sh.at[idx_d], gdx, sem_g),
            pltpu.async_copy(ys_sh.at[idx_d], gdy, sem_g),
            pltpu.async_copy(zs_sh.at[idx_d], gdz, sem_g),
        ]
        for d in gds:
            d.wait()
        ev = lax.fori_loop(0, _C // _L, grp, ev)
        sds = [
            pltpu.async_copy(ue, ae_sh.at[idx_s], sem_s, add=True),
            pltpu.async_copy(ue, ae_sh.at[idx_d], sem_s, add=True),
            pltpu.async_copy(usx, fx_sh.at[idx_s], sem_s, add=True),
            pltpu.async_copy(usy, fy_sh.at[idx_s], sem_s, add=True),
            pltpu.async_copy(usz, fz_sh.at[idx_s], sem_s, add=True),
            pltpu.async_copy(udx, fx_sh.at[idx_d], sem_s, add=True),
            pltpu.async_copy(udy, fy_sh.at[idx_d], sem_s, add=True),
            pltpu.async_copy(udz, fz_sh.at[idx_d], sem_s, add=True),
        ]
        for d in sds:
            d.wait()
        return ev

    evec = lax.fori_loop(0, n_chunks, chunk, jnp.zeros((_L,), jnp.float32))
    esum_v[...] = evec
    plsc.subcore_barrier()
    osl = pl.ds(cid * npad + r0, rpt)
    for sh, out in ((ae_sh, ae_out), (fx_sh, fx_out),
                    (fy_sh, fy_out), (fz_sh, fz_out)):
        pltpu.sync_copy(sh.at[sl], stage)
        pltpu.sync_copy(stage, out.at[osl])
    pltpu.sync_copy(esum_v, esum_out.at[pl.ds(wid * _L, _L)])


def kernel(positions, mapping):
    n = positions.shape[0]
    pad = _NPAD - n
    px = jnp.pad(positions[:, 0], (0, pad))
    py = jnp.pad(positions[:, 1], (0, pad))
    pz = jnp.pad(positions[:, 2], (0, pad))
    src = mapping[0]
    dst = mapping[1]

    mesh = plsc.VectorSubcoreMesh(core_axis_name="c", subcore_axis_name="s")
    f32 = jnp.float32
    kfn = pl.kernel(
        _sc_body,
        out_type=(
            jax.ShapeDtypeStruct((_NC * _NPAD,), f32),
            jax.ShapeDtypeStruct((_NC * _NPAD,), f32),
            jax.ShapeDtypeStruct((_NC * _NPAD,), f32),
            jax.ShapeDtypeStruct((_NC * _NPAD,), f32),
            jax.ShapeDtypeStruct((_NW * _L,), f32),
        ),
        mesh=mesh,
        scratch_types=(
            [pltpu.VMEM((_C,), jnp.int32)] * 2
            + [pltpu.VMEM((_C,), f32)] * 6
            + [pltpu.VMEM((_C,), f32)] * 7
            + [pltpu.VMEM((_L,), f32)]
            + [pltpu.VMEM((_NPAD // _NS,), f32)]
            + [pltpu.SemaphoreType.DMA] * 3
            + [pltpu.VMEM_SHARED((_NPAD,), f32)] * 7
        ),
    )
    ae, fx, fy, fz, esum = kfn(px, py, pz, src, dst)
    atom_energies = ae[:n] + ae[_NPAD:_NPAD + n]
    forces = jnp.stack(
        [fx[:n] + fx[_NPAD:_NPAD + n],
         fy[:n] + fy[_NPAD:_NPAD + n],
         fz[:n] + fz[_NPAD:_NPAD + n]], axis=-1)
    energy = 0.5 * jnp.sum(esum)
    return energy, atom_energies, forces


# speedup vs baseline: 86.4362x; 1.1809x over previous
import jax
import jax.numpy as jnp
from jax import lax
from jax.experimental import pallas as pl
from jax.experimental.pallas import tpu as pltpu
from jax.experimental.pallas import tpu_sc as plsc

_SIGMA = 1.0
_EPSILON = 5.0
_ALPHA = 5.0
_CUTOFF = 2.5

_NC = 2
_NS = 16
_NW = _NC * _NS
_L = 16
_C = 2000
_NPAD = 100096


def _rsqrt(d2):
    i = lax.bitcast_convert_type(d2, jnp.int32)
    i = jnp.int32(0x5F3759DF) - (i >> 1)
    y = lax.bitcast_convert_type(i, jnp.float32)
    for _ in range(3):
        y = y * (1.5 - 0.5 * d2 * y * y)
    return y


def _sc_body(px_hbm, py_hbm, pz_hbm, src_hbm, dst_hbm,
             ae_out, fx_out, fy_out, fz_out, esum_out,
             idx_s0, idx_d0, idx_s1, idx_d1,
             gsx0, gsy0, gsz0, gdx0, gdy0, gdz0,
             gsx1, gsy1, gsz1, gdx1, gdy1, gdz1,
             ue, usx, usy, usz, udx, udy, udz, esum_v, stage,
             sem_i0, sem_i1, sem_g0, sem_g1, sem_s,
             xs_sh, ys_sh, zs_sh, ae_sh, fx_sh, fy_sh, fz_sh):
    npad = px_hbm.shape[0]
    e = src_hbm.shape[0]
    cid = lax.axis_index("c")
    sid = lax.axis_index("s")
    wid = sid * _NC + cid
    rpt = npad // _NS
    r0 = sid * rpt

    sl = pl.ds(r0, rpt)

    def zfill(j, _):
        stage[pl.ds(j * _L, _L)] = jnp.zeros((_L,), jnp.float32)
        return 0

    lax.fori_loop(0, rpt // _L, zfill, 0)
    pltpu.sync_copy(stage, ae_sh.at[sl])
    pltpu.sync_copy(stage, fx_sh.at[sl])
    pltpu.sync_copy(stage, fy_sh.at[sl])
    pltpu.sync_copy(stage, fz_sh.at[sl])
    for hbm, sh in ((px_hbm, xs_sh), (py_hbm, ys_sh), (pz_hbm, zs_sh)):
        pltpu.sync_copy(hbm.at[sl], stage)
        pltpu.sync_copy(stage, sh.at[sl])
    plsc.subcore_barrier()

    e_per_w = e // _NW
    n_chunks = e_per_w // _C
    base0 = wid * e_per_w

    bufs = (
        (idx_s0, idx_d0, gsx0, gsy0, gsz0, gdx0, gdy0, gdz0, sem_i0, sem_g0),
        (idx_s1, idx_d1, gsx1, gsy1, gsz1, gdx1, gdy1, gdz1, sem_i1, sem_g1),
    )

    def issue_idx(i, b):
        i_s, i_d, sem_i = bufs[b][0], bufs[b][1], bufs[b][8]
        base = base0 + i * _C
        pltpu.async_copy(src_hbm.at[pl.ds(base, _C)], i_s, sem_i)
        pltpu.async_copy(dst_hbm.at[pl.ds(base, _C)], i_d, sem_i)

    def wait_idx(b):
        i_s, i_d, sem_i = bufs[b][0], bufs[b][1], bufs[b][8]
        pltpu.make_async_copy(src_hbm.at[pl.ds(0, _C)], i_s, sem_i).wait()
        pltpu.make_async_copy(dst_hbm.at[pl.ds(0, _C)], i_d, sem_i).wait()

    def issue_gathers(b):
        i_s, i_d = bufs[b][0], bufs[b][1]
        x1, y1, z1, x2, y2, z2 = bufs[b][2:8]
        sem_g = bufs[b][9]
        pltpu.async_copy(xs_sh.at[i_s], x1, sem_g)
        pltpu.async_copy(ys_sh.at[i_s], y1, sem_g)
        pltpu.async_copy(zs_sh.at[i_s], z1, sem_g)
        pltpu.async_copy(xs_sh.at[i_d], x2, sem_g)
        pltpu.async_copy(ys_sh.at[i_d], y2, sem_g)
        pltpu.async_copy(zs_sh.at[i_d], z2, sem_g)

    def wait_gathers(b):
        i_s, i_d = bufs[b][0], bufs[b][1]
        x1, y1, z1, x2, y2, z2 = bufs[b][2:8]
        sem_g = bufs[b][9]
        pltpu.make_async_copy(xs_sh.at[i_s], x1, sem_g).wait()
        pltpu.make_async_copy(ys_sh.at[i_s], y1, sem_g).wait()
        pltpu.make_async_copy(zs_sh.at[i_s], z1, sem_g).wait()
        pltpu.make_async_copy(xs_sh.at[i_d], x2, sem_g).wait()
        pltpu.make_async_copy(ys_sh.at[i_d], y2, sem_g).wait()
        pltpu.make_async_copy(zs_sh.at[i_d], z2, sem_g).wait()

    def make_grp(b):
        x1, y1, z1, x2, y2, z2 = bufs[b][2:8]

        def grp(g, ev):
            s = pl.ds(g * _L, _L)
            dx = x2[s] - x1[s]
            dy = y2[s] - y1[s]
            dz = z2[s] - z1[s]
            d2 = dx * dx + dy * dy + dz * dz
            valid = d2 > 0.0
            yk = _rsqrt(d2)
            dr = jnp.where(valid, d2 * yk, 0.0)
            em = jnp.exp(-_ALPHA * (dr - _SIGMA))
            om = 1.0 - em
            mask = valid & (dr < _CUTOFF)
            pe = jnp.where(mask, _EPSILON * om * om - _EPSILON, 0.0)
            pf = jnp.where(mask, (-2.0 * _ALPHA * _EPSILON) * em * om, 0.0)
            sc = pf * yk
            fx = sc * dx
            fy = sc * dy
            fz = sc * dz
            ue[s] = 0.5 * pe
            usx[s] = -fx
            usy[s] = -fy
            usz[s] = -fz
            udx[s] = fx
            udy[s] = fy
            udz[s] = fz
            return ev + pe

        return grp

    def scatter_chunk(b):
        i_s, i_d = bufs[b][0], bufs[b][1]
        sds = [
            pltpu.async_copy(ue, ae_sh.at[i_s], sem_s, add=True),
            pltpu.async_copy(ue, ae_sh.at[i_d], sem_s, add=True),
            pltpu.async_copy(usx, fx_sh.at[i_s], sem_s, add=True),
            pltpu.async_copy(usy, fy_sh.at[i_s], sem_s, add=True),
            pltpu.async_copy(usz, fz_sh.at[i_s], sem_s, add=True),
            pltpu.async_copy(udx, fx_sh.at[i_d], sem_s, add=True),
            pltpu.async_copy(udy, fy_sh.at[i_d], sem_s, add=True),
            pltpu.async_copy(udz, fz_sh.at[i_d], sem_s, add=True),
        ]
        for d in sds:
            d.wait()

    def phase(i, b, ev):
        wait_gathers(b)

        @pl.when(i + 1 < n_chunks)
        def _():
            issue_idx(i + 1, 1 - b)
            wait_idx(1 - b)
            issue_gathers(1 - b)

        ev = lax.fori_loop(0, _C // _L, make_grp(b), ev)
        scatter_chunk(b)
        return ev

    issue_idx(0, 0)
    wait_idx(0)
    issue_gathers(0)

    def pair(j, ev):
        ev = phase(2 * j, 0, ev)
        ev = phase(2 * j + 1, 1, ev)
        return ev

    evec = lax.fori_loop(0, n_chunks // 2, pair,
                         jnp.zeros((_L,), jnp.float32))
    esum_v[...] = evec
    plsc.subcore_barrier()
    osl = pl.ds(cid * npad + r0, rpt)
    for sh, out in ((ae_sh, ae_out), (fx_sh, fx_out),
                    (fy_sh, fy_out), (fz_sh, fz_out)):
        pltpu.sync_copy(sh.at[sl], stage)
        pltpu.sync_copy(stage, out.at[osl])
    pltpu.sync_copy(esum_v, esum_out.at[pl.ds(wid * _L, _L)])


def kernel(positions, mapping):
    n = positions.shape[0]
    pad = _NPAD - n
    px = jnp.pad(positions[:, 0], (0, pad))
    py = jnp.pad(positions[:, 1], (0, pad))
    pz = jnp.pad(positions[:, 2], (0, pad))
    src = mapping[0]
    dst = mapping[1]

    mesh = plsc.VectorSubcoreMesh(core_axis_name="c", subcore_axis_name="s")
    f32 = jnp.float32
    kfn = pl.kernel(
        _sc_body,
        out_type=(
            jax.ShapeDtypeStruct((_NC * _NPAD,), f32),
            jax.ShapeDtypeStruct((_NC * _NPAD,), f32),
            jax.ShapeDtypeStruct((_NC * _NPAD,), f32),
            jax.ShapeDtypeStruct((_NC * _NPAD,), f32),
            jax.ShapeDtypeStruct((_NW * _L,), f32),
        ),
        mesh=mesh,
        scratch_types=(
            [pltpu.VMEM((_C,), jnp.int32)] * 4
            + [pltpu.VMEM((_C,), f32)] * 12
            + [pltpu.VMEM((_C,), f32)] * 7
            + [pltpu.VMEM((_L,), f32)]
            + [pltpu.VMEM((_NPAD // _NS,), f32)]
            + [pltpu.SemaphoreType.DMA] * 5
            + [pltpu.VMEM_SHARED((_NPAD,), f32)] * 7
        ),
    )
    ae, fx, fy, fz, esum = kfn(px, py, pz, src, dst)
    atom_energies = ae[:n] + ae[_NPAD:_NPAD + n]
    forces = jnp.stack(
        [fx[:n] + fx[_NPAD:_NPAD + n],
         fy[:n] + fy[_NPAD:_NPAD + n],
         fz[:n] + fz[_NPAD:_NPAD + n]], axis=-1)
    energy = 0.5 * jnp.sum(esum)
    return energy, atom_energies, forces
